# baseline (speedup 1.0000x reference)
import functools

import jax
import jax.numpy as jnp
from jax import lax
from jax.experimental import pallas as pl
from jax.experimental.pallas import tpu as pltpu
from jax.experimental.pallas import tpu_sc as plsc

_CHUNK = 128
_NBUF = 8


@functools.partial(jax.jit, static_argnames=("nc", "ns"))
def _sc_gather(ids_2d, pe, nc, ns):
    n_chunks_total, chunk = ids_2d.shape
    _, d = pe.shape
    nw = nc * ns
    n_chunks = n_chunks_total // nw
    n_outer = n_chunks // _NBUF
    assert n_chunks_total == nw * n_outer * _NBUF

    mesh = plsc.VectorSubcoreMesh(
        core_axis_name="c", subcore_axis_name="s", num_cores=nc, num_subcores=ns
    )

    @functools.partial(
        pl.kernel,
        out_type=jax.ShapeDtypeStruct((n_chunks_total * chunk, d), jnp.float32),
        mesh=mesh,
        scratch_types=[
            pltpu.VMEM((n_chunks, chunk), jnp.int32),
            pltpu.VMEM((_NBUF, chunk, d), jnp.float32),
            pltpu.SemaphoreType.DMA((_NBUF,)),
            pltpu.SemaphoreType.DMA((_NBUF,)),
        ],
        compiler_params=pltpu.CompilerParams(use_tc_tiling_on_sc=False),
    )
    def k(ids_hbm, pe_hbm, out_hbm, idx_v, rows_v, gsem, ssem):
        cid = lax.axis_index("c")
        sid = lax.axis_index("s")
        wid = sid * nc + cid
        cbase = wid * n_chunks

        pltpu.sync_copy(ids_hbm.at[pl.ds(cbase, n_chunks)], idx_v)

        def gather_start(j, b):
            pltpu.async_copy(pe_hbm.at[idx_v.at[j]], rows_v.at[b], gsem.at[b])

        def gather_wait(b):
            pltpu.make_async_copy(
                pe_hbm.at[pl.ds(0, chunk)], rows_v.at[b], gsem.at[b]
            ).wait()

        def scatter_start(j, b):
            pltpu.async_copy(
                rows_v.at[b], out_hbm.at[pl.ds((cbase + j) * chunk, chunk)], ssem.at[b]
            )

        def scatter_wait(b):
            pltpu.make_async_copy(
                rows_v.at[b], out_hbm.at[pl.ds(0, chunk)], ssem.at[b]
            ).wait()

        for b in range(_NBUF):
            gather_start(b, b)

        def outer(g, carry):
            for b in range(_NBUF):
                gather_wait(b)
                scatter_start(g * _NBUF + b, b)
            for b in range(_NBUF):
                scatter_wait(b)
                gather_start((g + 1) * _NBUF + b, b)
            return carry

        lax.fori_loop(0, n_outer - 1, outer, 0, unroll=False)

        g_last = n_outer - 1
        for b in range(_NBUF):
            gather_wait(b)
            scatter_start(g_last * _NBUF + b, b)
        for b in range(_NBUF):
            scatter_wait(b)

    return k(ids_2d, pe)


def kernel(ids, pe):
    b, s = ids.shape
    v, d = pe.shape
    total = b * s
    info = plsc.get_sparse_core_info()
    nc, ns = info.num_cores, info.num_subcores
    grain = nc * ns * _CHUNK * _NBUF
    total_pad = ((total + grain - 1) // grain) * grain
    ids_flat = ids.reshape(total).astype(jnp.int32)
    if total_pad != total:
        ids_flat = jnp.pad(ids_flat, (0, total_pad - total))
    rows = _sc_gather(ids_flat.reshape(total_pad // _CHUNK, _CHUNK), pe, nc, ns)
    if total_pad != total:
        rows = rows[:total]
    return rows.reshape(b, s, d)


# speedup vs baseline: 1.0007x; 1.0007x over previous
import functools

import jax
import jax.numpy as jnp
from jax import lax
from jax.experimental import pallas as pl
from jax.experimental.pallas import tpu as pltpu
from jax.experimental.pallas import tpu_sc as plsc

_CHUNK = 100
_NBUF = 8


@functools.partial(jax.jit, static_argnames=("nc", "ns"))
def _sc_gather(ids_2d, pe, nc, ns):
    n_chunks_total, chunk = ids_2d.shape
    _, d = pe.shape
    s_len = 2 * chunk
    b_total = n_chunks_total // 2
    nw = nc * ns
    n_chunks = n_chunks_total // nw
    n_outer = n_chunks // _NBUF
    assert n_chunks_total == nw * n_outer * _NBUF

    mesh = plsc.VectorSubcoreMesh(
        core_axis_name="c", subcore_axis_name="s", num_cores=nc, num_subcores=ns
    )

    @functools.partial(
        pl.kernel,
        out_type=jax.ShapeDtypeStruct((b_total, s_len, d), jnp.float32),
        mesh=mesh,
        scratch_types=[
            pltpu.VMEM((n_chunks, chunk), jnp.int32),
            pltpu.VMEM((_NBUF, chunk, d), jnp.float32),
            pltpu.SemaphoreType.DMA((_NBUF,)),
            pltpu.SemaphoreType.DMA((_NBUF,)),
        ],
        compiler_params=pltpu.CompilerParams(use_tc_tiling_on_sc=False),
    )
    def k(ids_hbm, pe_hbm, out_hbm, idx_v, rows_v, gsem, ssem):
        cid = lax.axis_index("c")
        sid = lax.axis_index("s")
        wid = sid * nc + cid
        cbase = wid * n_chunks

        pltpu.sync_copy(ids_hbm.at[pl.ds(cbase, n_chunks)], idx_v)

        def gather_start(j, b):
            pltpu.async_copy(pe_hbm.at[idx_v.at[j]], rows_v.at[b], gsem.at[b])

        def gather_wait(b):
            pltpu.make_async_copy(
                pe_hbm.at[pl.ds(0, chunk)], rows_v.at[b], gsem.at[b]
            ).wait()

        def scatter_start(j, b):
            c = cbase + j
            pltpu.async_copy(
                rows_v.at[b],
                out_hbm.at[c // 2, pl.ds((c % 2) * chunk, chunk)],
                ssem.at[b],
            )

        def scatter_wait(b):
            pltpu.make_async_copy(
                rows_v.at[b], out_hbm.at[0, pl.ds(0, chunk)], ssem.at[b]
            ).wait()

        for b in range(_NBUF):
            gather_start(b, b)

        def outer(g, carry):
            for b in range(_NBUF):
                gather_wait(b)
                scatter_start(g * _NBUF + b, b)
            for b in range(_NBUF):
                scatter_wait(b)
                gather_start((g + 1) * _NBUF + b, b)
            return carry

        lax.fori_loop(0, n_outer - 1, outer, 0, unroll=False)

        g_last = n_outer - 1
        for b in range(_NBUF):
            gather_wait(b)
            scatter_start(g_last * _NBUF + b, b)
        for b in range(_NBUF):
            scatter_wait(b)

    return k(ids_2d, pe)


def kernel(ids, pe):
    b, s = ids.shape
    v, d = pe.shape
    info = plsc.get_sparse_core_info()
    nc, ns = info.num_cores, info.num_subcores
    ids_2d = ids.reshape(b * s // _CHUNK, _CHUNK).astype(jnp.int32)
    return _sc_gather(ids_2d, pe, nc, ns)


# speedup vs baseline: 1.2212x; 1.2204x over previous
import functools

import jax
import jax.numpy as jnp
from jax import lax
from jax.experimental import pallas as pl
from jax.experimental.pallas import tpu as pltpu
from jax.experimental.pallas import tpu_sc as plsc

_CHUNK = 128
_NBUF = 4


@functools.partial(jax.jit, static_argnames=("nc", "ns"))
def _sc_gather(ids_2d, pe_pad, nc, ns):
    n_chunks_total, chunk = ids_2d.shape
    _, d = pe_pad.shape
    nw = nc * ns
    n_chunks = n_chunks_total // nw
    n_outer = n_chunks // _NBUF
    assert n_chunks_total == nw * n_outer * _NBUF

    mesh = plsc.VectorSubcoreMesh(
        core_axis_name="c", subcore_axis_name="s", num_cores=nc, num_subcores=ns
    )

    @functools.partial(
        pl.kernel,
        out_type=jax.ShapeDtypeStruct((n_chunks_total * chunk, d), jnp.float32),
        mesh=mesh,
        scratch_types=[
            pltpu.VMEM((n_chunks, chunk), jnp.int32),
            pltpu.VMEM((_NBUF, chunk, d), jnp.float32),
            pltpu.SemaphoreType.DMA((_NBUF,)),
            pltpu.SemaphoreType.DMA((_NBUF,)),
        ],
        compiler_params=pltpu.CompilerParams(use_tc_tiling_on_sc=False),
    )
    def k(ids_hbm, pe_hbm, out_hbm, idx_v, rows_v, gsem, ssem):
        cid = lax.axis_index("c")
        sid = lax.axis_index("s")
        wid = sid * nc + cid
        cbase = wid * n_chunks

        pltpu.sync_copy(ids_hbm.at[pl.ds(cbase, n_chunks)], idx_v)

        def gather_start(j, b):
            pltpu.async_copy(pe_hbm.at[idx_v.at[j]], rows_v.at[b], gsem.at[b])

        def gather_wait(b):
            pltpu.make_async_copy(
                pe_hbm.at[pl.ds(0, chunk)], rows_v.at[b], gsem.at[b]
            ).wait()

        def scatter_start(j, b):
            pltpu.async_copy(
                rows_v.at[b], out_hbm.at[pl.ds((cbase + j) * chunk, chunk)], ssem.at[b]
            )

        def scatter_wait(b):
            pltpu.make_async_copy(
                rows_v.at[b], out_hbm.at[pl.ds(0, chunk)], ssem.at[b]
            ).wait()

        for b in range(_NBUF):
            gather_start(b, b)

        def outer(g, carry):
            for b in range(_NBUF):
                gather_wait(b)
                scatter_start(g * _NBUF + b, b)
            for b in range(_NBUF):
                scatter_wait(b)
                gather_start((g + 1) * _NBUF + b, b)
            return carry

        lax.fori_loop(0, n_outer - 1, outer, 0, unroll=False)

        g_last = n_outer - 1
        for b in range(_NBUF):
            gather_wait(b)
            scatter_start(g_last * _NBUF + b, b)
        for b in range(_NBUF):
            scatter_wait(b)

    return k(ids_2d, pe_pad)


def kernel(ids, pe):
    b, s = ids.shape
    v, d = pe.shape
    info = plsc.get_sparse_core_info()
    nc, ns = info.num_cores, info.num_subcores
    ids_2d = ids.reshape(b * s // _CHUNK, _CHUNK).astype(jnp.int32)
    pe_pad = jnp.pad(pe, ((0, 0), (0, 128 - d)))
    rows = _sc_gather(ids_2d, pe_pad, nc, ns)
    return rows[:, :d].reshape(b, s, d)
